# baseline (speedup 1.0000x reference)
import functools

import jax
import jax.numpy as jnp
from jax import lax
from jax.experimental import pallas as pl
from jax.experimental.pallas import tpu as pltpu
from jax.experimental.pallas import tpu_sc as plsc

_NT = 2048
_NCH = 131
_CELL = 3 * _NCH
_ANCHORS = ((0.5, 1.0, 2.0), (2.0, 4.0, 6.0), (6.0, 10.0, 20.0))
_GRIDS = (1024, 512, 256)
_DUR = 60.0


def _sc_gather(sm_t, md_t, lg_t, b_col, c_col):
    info = plsc.get_sparse_core_info()
    nw = info.num_cores * info.num_subcores
    bpw = _NT // nw
    nchunk = bpw // 16
    mesh = plsc.VectorSubcoreMesh(core_axis_name="c", subcore_axis_name="s")

    @functools.partial(
        pl.kernel,
        mesh=mesh,
        out_type=[jax.ShapeDtypeStruct((_NT, _CELL), jnp.float32)
                  for _ in range(3)],
        scratch_types=[
            pltpu.VMEM((bpw,), jnp.float32),
            pltpu.VMEM((bpw,), jnp.float32),
            pltpu.VMEM((bpw,), jnp.int32),
            pltpu.VMEM((bpw, _CELL), jnp.float32),
            pltpu.SemaphoreType.DMA,
        ],
        compiler_params=pltpu.CompilerParams(use_tc_tiling_on_sc=False),
    )
    def k(sm_hbm, md_hbm, lg_hbm, b_hbm, c_hbm, o_sm, o_md, o_lg,
          b_v, c_v, idx_v, rows_v, sem):
        wid = lax.axis_index("s") * info.num_cores + lax.axis_index("c")
        base = wid * bpw
        pltpu.sync_copy(b_hbm.at[pl.ds(base, bpw)], b_v)
        pltpu.sync_copy(c_hbm.at[pl.ds(base, bpw)], c_v)
        for tab, oh, g in ((sm_hbm, o_sm, _GRIDS[0]),
                           (md_hbm, o_md, _GRIDS[1]),
                           (lg_hbm, o_lg, _GRIDS[2])):
            for c in range(nchunk):
                bvec = b_v[pl.ds(c * 16, 16)]
                cvec = c_v[pl.ds(c * 16, 16)]
                gi = (cvec / _DUR * g).astype(jnp.int32)
                gi = jnp.minimum(jnp.maximum(gi, 0), g - 1)
                idx_v[pl.ds(c * 16, 16)] = bvec.astype(jnp.int32) * g + gi
            pltpu.async_copy(tab.at[idx_v], rows_v, sem).wait()
            pltpu.sync_copy(rows_v, oh.at[pl.ds(base, bpw)])

    return k(sm_t, md_t, lg_t, b_col, c_col)


def _conf_body(sm_ref, md_ref, lg_ref, out_ref):
    i = pl.program_id(0)

    @pl.when(i == 0)
    def _():
        out_ref[...] = jnp.zeros_like(out_ref)

    e0 = (lax.broadcasted_iota(jnp.int32, (_NCH, 1), 0) == 0
          ).astype(jnp.float32)
    parts = []
    for ref in (sm_ref, md_ref, lg_ref):
        x = jnp.dot(ref[0], e0, preferred_element_type=jnp.float32)
        sp = jnp.maximum(x, 0.0) + jnp.log1p(jnp.exp(-jnp.abs(x)))
        parts.append(jnp.sum(sp, keepdims=True).reshape(1, 1))
    out_ref[...] += jnp.concatenate(parts, axis=1)


def _conf_sums(sm, md, lg):
    nb = sm.shape[0]
    ins = [p.reshape(p.shape[0], p.shape[1] * 3, _NCH) for p in (sm, md, lg)]
    return pl.pallas_call(
        _conf_body,
        grid=(nb,),
        in_specs=[pl.BlockSpec((1, a.shape[1], _NCH), lambda i: (i, 0, 0))
                  for a in ins],
        out_specs=pl.BlockSpec((1, 3), lambda i: (0, 0)),
        out_shape=jax.ShapeDtypeStruct((1, 3), jnp.float32),
    )(*ins)


_ATAN_C = (9.999999055342e-01, -3.333265778287e-01, 1.998653634321e-01,
           -1.416432519127e-01, 1.050728938609e-01, -7.247887440757e-02,
           3.989881857950e-02, -1.445823919594e-02, 2.468130925107e-03)


def _atan(x):
    s = jnp.abs(x)
    inv = s > 1.0
    z = jnp.where(inv, 1.0 / jnp.maximum(s, 1e-30), s)
    z2 = z * z
    p = jnp.full_like(z, _ATAN_C[-1])
    for c in _ATAN_C[-2::-1]:
        p = p * z2 + c
    p = p * z
    r = jnp.where(inv, jnp.float32(jnp.pi / 2) - p, p)
    return jnp.sign(x) * r


def _sparse_body(t_ref, gsm_ref, gmd_ref, glg_ref, out_ref):
    t = t_ref[...]
    cls = t[:, 1:2].astype(jnp.int32)
    c_t = t[:, 2:3]
    w_t = t[:, 3:4]
    onehot = (cls == lax.broadcasted_iota(jnp.int32, (1, 128), 1)
              ).astype(jnp.float32)
    e = 1e-15
    tx1 = c_t - w_t / 2
    tx2 = c_t + w_t / 2
    at_t = _atan(w_t / 10.0)
    rows = []
    for g_ref, ancs in ((gsm_ref, _ANCHORS[0]), (gmd_ref, _ANCHORS[1]),
                        (glg_ref, _ANCHORS[2])):
        box_s = corr_s = cls_s = cnt_s = None
        for a, anc in enumerate(ancs):
            cell = g_ref[:, a * _NCH:(a + 1) * _NCH]
            xconf = cell[:, 0:1]
            clsl = cell[:, 1:129]
            pc = cell[:, 129:130]
            pw = cell[:, 130:131]
            r = w_t / anc
            valid = (jnp.maximum(r, 1.0 / r) < 4.0).astype(jnp.float32)
            px1 = pc - pw / 2
            px2 = pc + pw / 2
            inter = jnp.maximum(
                jnp.minimum(px2, tx2) - jnp.maximum(px1, tx1), 0.0) * 10.0
            union = pw * 10.0 + w_t * 10.0 - inter
            iou = inter / (union + e)
            cw_ = jnp.maximum(px2, tx2) - jnp.minimum(px1, tx1)
            c2 = cw_ * cw_ + (100.0 + e)
            dat = at_t - _atan(pw / 10.0)
            v = (4.0 / (jnp.pi ** 2)) * dat * dat
            a_trm = v / (1.0 + e - iou) + v
            rho2 = (pc - c_t) * (pc - c_t)
            ciou = jnp.maximum(iou - (rho2 / c2 + a_trm * v), 0.0)
            m = jnp.max(clsl, axis=1, keepdims=True)
            lse = jnp.log(jnp.sum(jnp.exp(clsl - m), axis=1,
                                  keepdims=True)) + m
            picked = jnp.sum(clsl * onehot, axis=1, keepdims=True)
            box = jnp.sum(valid * (1.0 - ciou), keepdims=True)
            cor = jnp.sum(valid * xconf * ciou, keepdims=True)
            cl = jnp.sum(valid * (lse - picked), keepdims=True)
            cn = jnp.sum(valid, keepdims=True)
            if box_s is None:
                box_s, corr_s, cls_s, cnt_s = box, cor, cl, cn
            else:
                box_s, corr_s, cls_s, cnt_s = (
                    box_s + box, corr_s + cor, cls_s + cl, cnt_s + cn)
        rows.append(jnp.concatenate([box_s, corr_s, cls_s, cnt_s], axis=1))
    out_ref[...] = jnp.concatenate(rows, axis=0)


def _sparse_stats(targets, g_sm, g_md, g_lg):
    return pl.pallas_call(
        _sparse_body,
        out_shape=jax.ShapeDtypeStruct((3, 4), jnp.float32),
    )(targets, g_sm, g_md, g_lg)


def kernel(sm_preds, md_preds, lg_preds, targets):
    tabs = [p.reshape(p.shape[0] * p.shape[1], _CELL)
            for p in (sm_preds, md_preds, lg_preds)]
    g_sm, g_md, g_lg = _sc_gather(*tabs, targets[:, 0] + 0.0,
                                  targets[:, 2] + 0.0)
    sp = _conf_sums(sm_preds, md_preds, lg_preds)[0]
    st = _sparse_stats(targets, g_sm, g_md, g_lg)
    conf_w = jnp.asarray([4.0, 1.0, 0.4], jnp.float32)
    nconf = jnp.asarray([p.shape[0] * p.shape[1] * 3 for p in
                         (sm_preds, md_preds, lg_preds)], jnp.float32)
    cnt = st[:, 3]
    safe = cnt > 0
    lbox = jnp.sum(jnp.where(safe, st[:, 0] / cnt, 0.0))
    lcls = jnp.sum(jnp.where(safe, st[:, 2] / cnt, 0.0))
    lconf = jnp.sum(conf_w * (sp - st[:, 1]) / nconf)
    return (lbox + lconf + lcls) * jnp.float32(sm_preds.shape[0])


# speedup vs baseline: 3.8752x; 3.8752x over previous
import functools

import jax
import jax.numpy as jnp
from jax import lax
from jax.experimental import pallas as pl
from jax.experimental.pallas import tpu as pltpu
from jax.experimental.pallas import tpu_sc as plsc

_NT = 2048
_NCH = 131
_ANCHORS = ((0.5, 1.0, 2.0), (2.0, 4.0, 6.0), (6.0, 10.0, 20.0))
_GRIDS = (1024, 512, 256)
_DUR = 60.0


def _dense_body(sm_ref, md_ref, lg_ref, esm_ref, emd_ref, elg_ref, cf_ref):
    a = pl.program_id(0)
    c = pl.program_id(1)

    @pl.when(jnp.logical_and(a == 0, c == 0))
    def _():
        cf_ref[...] = jnp.zeros_like(cf_ref)

    col = lax.broadcasted_iota(jnp.int32, (3, 3), 1)
    refs = ((sm_ref, esm_ref), (md_ref, emd_ref), (lg_ref, elg_ref))
    for x_ref, e_ref in refs:
        @pl.when(c == 1)
        def _(x_ref=x_ref, e_ref=e_ref):
            e_ref[...] = jnp.exp(x_ref[...])

        @pl.when(c > 1)
        def _(x_ref=x_ref, e_ref=e_ref):
            e_ref[...] += jnp.exp(x_ref[...])

    @pl.when(c == 0)
    def _():
        vals = []
        for x_ref, _e in refs:
            x = x_ref[...]
            sp = jnp.maximum(x, 0.0) + jnp.log1p(jnp.exp(-jnp.abs(x)))
            vals.append(jnp.sum(sp, keepdims=True).reshape(1, 1))
        cf_ref[...] += jnp.concatenate(vals, axis=0) * (col == a)


def _dense_pass(p2s):
    grids = _GRIDS
    out_shapes = ([jax.ShapeDtypeStruct((192, g), jnp.float32) for g in grids]
                  + [jax.ShapeDtypeStruct((3, 3), jnp.float32)])
    in_specs = [pl.BlockSpec((64, g), lambda a, c: (a * 131 + c, 0))
                for g in grids]
    out_specs = ([pl.BlockSpec((64, g), lambda a, c: (a, 0)) for g in grids]
                 + [pl.BlockSpec((3, 3), lambda a, c: (0, 0))])
    return pl.pallas_call(
        _dense_body,
        grid=(3, 129),
        in_specs=in_specs,
        out_specs=out_specs,
        out_shape=out_shapes,
    )(*p2s)


def _sc_gather(xs_sm, xs_md, xs_lg, e_sm, e_md, e_lg, b_col, c_col, cls_col):
    info = plsc.get_sparse_core_info()
    nw = info.num_cores * info.num_subcores
    bpw = _NT // nw
    nchunk = bpw // 16
    mesh = plsc.VectorSubcoreMesh(core_axis_name="c", subcore_axis_name="s")

    @functools.partial(
        pl.kernel,
        mesh=mesh,
        out_type=[jax.ShapeDtypeStruct((nw, 768), jnp.float32)
                  for _ in range(3)]
        + [jax.ShapeDtypeStruct((nw, 192), jnp.float32) for _ in range(3)],
        scratch_types=[
            pltpu.VMEM((bpw,), jnp.float32),
            pltpu.VMEM((bpw,), jnp.float32),
            pltpu.VMEM((bpw,), jnp.float32),
            pltpu.VMEM((768,), jnp.int32),
            pltpu.VMEM((192,), jnp.int32),
            pltpu.VMEM((768,), jnp.float32),
            pltpu.VMEM((192,), jnp.float32),
            pltpu.SemaphoreType.DMA,
        ],
    )
    def k(xsm, xmd, xlg, esm, emd, elg, bh, ch, clh,
          gx_sm, gx_md, gx_lg, ge_sm, ge_md, ge_lg,
          b_v, c_v, cl_v, idx_v, eidx_v, dst_v, edst_v, sem):
        wid = lax.axis_index("s") * info.num_cores + lax.axis_index("c")
        base = wid * bpw
        pltpu.sync_copy(bh.at[pl.ds(base, bpw)], b_v)
        pltpu.sync_copy(ch.at[pl.ds(base, bpw)], c_v)
        pltpu.sync_copy(clh.at[pl.ds(base, bpw)], cl_v)
        for xs, es, gx, ge, g in ((xsm, esm, gx_sm, ge_sm, _GRIDS[0]),
                                  (xmd, emd, gx_md, ge_md, _GRIDS[1]),
                                  (xlg, elg, gx_lg, ge_lg, _GRIDS[2])):
            ps = 64 * g
            for c4 in range(nchunk):
                bvec = b_v[pl.ds(c4 * 16, 16)]
                cvec = c_v[pl.ds(c4 * 16, 16)]
                clvec = cl_v[pl.ds(c4 * 16, 16)]
                gi = (cvec / _DUR * g).astype(jnp.int32)
                gi = jnp.minimum(jnp.maximum(gi, 0), g - 1)
                pos = bvec.astype(jnp.int32) * g + gi
                cls_off = clvec.astype(jnp.int32) * ps
                for a in range(3):
                    sl = lambda v: pl.ds((v * 3 + a) * 64 + c4 * 16, 16)
                    idx_v[sl(0)] = pos + (a * 131) * ps
                    idx_v[sl(1)] = pos + (a * 131 + 129) * ps
                    idx_v[sl(2)] = pos + (a * 131 + 130) * ps
                    idx_v[sl(3)] = pos + (a * 131 + 1) * ps + cls_off
                    eidx_v[pl.ds(a * 64 + c4 * 16, 16)] = pos + a * ps
            for kk in range(6):
                pltpu.async_copy(
                    xs.at[idx_v.at[pl.ds(kk * 128, 128)]],
                    dst_v.at[pl.ds(kk * 128, 128)], sem).wait()
            pltpu.async_copy(es.at[eidx_v.at[pl.ds(0, 128)]],
                             edst_v.at[pl.ds(0, 128)], sem).wait()
            pltpu.async_copy(es.at[eidx_v.at[pl.ds(128, 64)]],
                             edst_v.at[pl.ds(128, 64)], sem).wait()
            pltpu.sync_copy(dst_v, gx.at[wid])
            pltpu.sync_copy(edst_v, ge.at[wid])

    return k(xs_sm, xs_md, xs_lg, e_sm, e_md, e_lg, b_col, c_col, cls_col)


_ATAN_C = (9.999999055342e-01, -3.333265778287e-01, 1.998653634321e-01,
           -1.416432519127e-01, 1.050728938609e-01, -7.247887440757e-02,
           3.989881857950e-02, -1.445823919594e-02, 2.468130925107e-03)


def _atan(x):
    s = jnp.abs(x)
    inv = s > 1.0
    z = jnp.where(inv, 1.0 / jnp.maximum(s, 1e-30), s)
    z2 = z * z
    p = jnp.full_like(z, _ATAN_C[-1])
    for c in _ATAN_C[-2::-1]:
        p = p * z2 + c
    p = p * z
    r = jnp.where(inv, jnp.float32(jnp.pi / 2) - p, p)
    return jnp.sign(x) * r


def _sparse_body(tc_ref, tw_ref, gsm_ref, gmd_ref, glg_ref,
                 esm_ref, emd_ref, elg_ref, out_ref):
    c_t = tc_ref[...]
    w_t = tw_ref[...]
    e = 1e-15
    tx1 = c_t - w_t / 2
    tx2 = c_t + w_t / 2
    at_t = _atan(w_t / 10.0)
    rows = []
    for g_ref, e_ref, ancs in ((gsm_ref, esm_ref, _ANCHORS[0]),
                               (gmd_ref, emd_ref, _ANCHORS[1]),
                               (glg_ref, elg_ref, _ANCHORS[2])):
        gx = g_ref[...]
        ge = e_ref[...]
        box_s = corr_s = cls_s = cnt_s = None
        for a, anc in enumerate(ancs):
            xconf = gx[:, (0 * 3 + a) * 64:(0 * 3 + a) * 64 + 64]
            pc = gx[:, (1 * 3 + a) * 64:(1 * 3 + a) * 64 + 64]
            pw = gx[:, (2 * 3 + a) * 64:(2 * 3 + a) * 64 + 64]
            picked = gx[:, (3 * 3 + a) * 64:(3 * 3 + a) * 64 + 64]
            esum = ge[:, a * 64:a * 64 + 64]
            r = w_t / anc
            valid = (jnp.maximum(r, 1.0 / r) < 4.0).astype(jnp.float32)
            px1 = pc - pw / 2
            px2 = pc + pw / 2
            inter = jnp.maximum(
                jnp.minimum(px2, tx2) - jnp.maximum(px1, tx1), 0.0) * 10.0
            union = pw * 10.0 + w_t * 10.0 - inter
            iou = inter / (union + e)
            cw_ = jnp.maximum(px2, tx2) - jnp.minimum(px1, tx1)
            c2 = cw_ * cw_ + (100.0 + e)
            dat = at_t - _atan(pw / 10.0)
            v = (4.0 / (jnp.pi ** 2)) * dat * dat
            a_trm = v / (1.0 + e - iou) + v
            rho2 = (pc - c_t) * (pc - c_t)
            ciou = jnp.maximum(iou - (rho2 / c2 + a_trm * v), 0.0)
            lse = jnp.log(esum)
            box = jnp.sum(valid * (1.0 - ciou), keepdims=True)
            cor = jnp.sum(valid * xconf * ciou, keepdims=True)
            cl = jnp.sum(valid * (lse - picked), keepdims=True)
            cn = jnp.sum(valid, keepdims=True)
            if box_s is None:
                box_s, corr_s, cls_s, cnt_s = box, cor, cl, cn
            else:
                box_s, corr_s, cls_s, cnt_s = (
                    box_s + box, corr_s + cor, cls_s + cl, cnt_s + cn)
        rows.append(jnp.concatenate([box_s, corr_s, cls_s, cnt_s], axis=1))
    out_ref[...] = jnp.concatenate(rows, axis=0)


def _sparse_stats(tc2, tw2, g_sm, g_md, g_lg, e_sm, e_md, e_lg):
    return pl.pallas_call(
        _sparse_body,
        out_shape=jax.ShapeDtypeStruct((3, 4), jnp.float32),
    )(tc2, tw2, g_sm, g_md, g_lg, e_sm, e_md, e_lg)


def kernel(sm_preds, md_preds, lg_preds, targets):
    p2s = [jnp.transpose(p, (2, 3, 0, 1)).reshape(3 * _NCH * 64, p.shape[1])
           for p in (sm_preds, md_preds, lg_preds)]
    e_sm, e_md, e_lg, cf = _dense_pass(p2s)
    xs = [p.reshape(-1) for p in p2s]
    es = [e.reshape(-1) for e in (e_sm, e_md, e_lg)]
    g_sm, g_md, g_lg, ge_sm, ge_md, ge_lg = _sc_gather(
        *xs, *es, targets[:, 0] + 0.0, targets[:, 2] + 0.0,
        targets[:, 1] + 0.0)
    tc2 = targets[:, 2].reshape(32, 64)
    tw2 = targets[:, 3].reshape(32, 64)
    st = _sparse_stats(tc2, tw2, g_sm, g_md, g_lg, ge_sm, ge_md, ge_lg)
    conf_w = jnp.asarray([4.0, 1.0, 0.4], jnp.float32)
    nconf = jnp.asarray([64 * g * 3 for g in _GRIDS], jnp.float32)
    sp = jnp.sum(cf, axis=1)
    cnt = st[:, 3]
    safe = cnt > 0
    lbox = jnp.sum(jnp.where(safe, st[:, 0] / cnt, 0.0))
    lcls = jnp.sum(jnp.where(safe, st[:, 2] / cnt, 0.0))
    lconf = jnp.sum(conf_w * (sp - st[:, 1]) / nconf)
    return (lbox + lconf + lcls) * jnp.float32(sm_preds.shape[0])
